# Initial kernel scaffold; baseline (speedup 1.0000x reference)
import jax
import jax.numpy as jnp
from jax.experimental import pallas as pl


def kernel(hidden_states, gate_w, w1, w3, w2):
    raise NotImplementedError("write your pallas kernel here")


# speedup vs baseline: 1.3389x; 1.3389x over previous
import functools

import jax
import jax.numpy as jnp
from jax import lax
from jax.experimental import pallas as pl
from jax.experimental.pallas import tpu as pltpu
from jax.experimental.pallas import tpu_sc as plsc

TOPK = 2
TILE_M = 256
TILE_SHIFT = 8
NUM_TILES = 24
NSLOT = NUM_TILES * TILE_M
SCAN_CHUNK = 512


def _router_body(x_ref, gw_ref, pos_ref, rw_ref, ex_ref):
    x = x_ref[...]
    gw = gw_ref[...]
    E = gw.shape[0]
    T = x.shape[0]
    logits = lax.dot_general(gw, x, (((1,), (1,)), ((), ())),
                             preferred_element_type=jnp.float32)
    row = lax.broadcasted_iota(jnp.int32, (E, T), 0)
    v0 = jnp.max(logits, axis=0, keepdims=True)
    a0 = jnp.min(jnp.where(logits == v0, row, E), axis=0, keepdims=True)
    masked = jnp.where(row == a0, -jnp.inf, logits)
    v1 = jnp.max(masked, axis=0, keepdims=True)
    a1 = jnp.min(jnp.where(masked == v1, row, E), axis=0, keepdims=True)
    d = jnp.exp(v1 - v0)
    w0 = 1.0 / (1.0 + d)
    w1 = d / (1.0 + d)

    oh0 = (row == a0).astype(jnp.float32)
    oh1 = (row == a1).astype(jnp.float32)

    C = SCAN_CHUNK
    ci = lax.broadcasted_iota(jnp.int32, (C, C), 0)
    cj = lax.broadcasted_iota(jnp.int32, (C, C), 1)
    upper_incl = (ci <= cj).astype(jnp.float32)
    carry = jnp.zeros((E, 1), jnp.float32)
    ranks = []
    for oh in (oh0, oh1):
        chunks = []
        for c in range(T // C):
            ohc = oh[:, c * C:(c + 1) * C]
            run = lax.dot_general(ohc, upper_incl, (((1,), (0,)), ((), ())),
                                  preferred_element_type=jnp.float32) + carry
            chunks.append(jnp.sum(run * ohc, axis=0, keepdims=True))
            carry = run[:, C - 1:C]
        ranks.append(jnp.concatenate(chunks, axis=1) - 1.0)
    counts = carry

    counts_i = counts.astype(jnp.int32)
    tiles = lax.shift_right_logical(counts_i + (TILE_M - 1), TILE_SHIFT)
    tiles_f = tiles.astype(jnp.float32)
    ei = lax.broadcasted_iota(jnp.int32, (E, E), 0)
    ej = lax.broadcasted_iota(jnp.int32, (E, E), 1)
    strict_lower = (ej < ei).astype(jnp.float32)
    tbase = lax.dot_general(strict_lower, tiles_f, (((1,), (0,)), ((), ())),
                            preferred_element_type=jnp.float32)
    pbase = tbase * float(TILE_M)

    pos0 = jnp.sum(oh0 * pbase, axis=0, keepdims=True) + ranks[0]
    pos1 = jnp.sum(oh1 * pbase, axis=0, keepdims=True) + ranks[1]
    pos_ref[0:1, :] = pos0.astype(jnp.int32)
    pos_ref[1:2, :] = pos1.astype(jnp.int32)
    rw_ref[0:1, :] = w0
    rw_ref[1:2, :] = w1

    g = lax.broadcasted_iota(jnp.int32, (1, 32), 1)
    owner = jnp.sum((tbase <= g.astype(jnp.float32)).astype(jnp.float32),
                    axis=0, keepdims=True).astype(jnp.int32) - 1
    total = jnp.sum(tiles_f).astype(jnp.int32)
    ex_ref[...] = jnp.where(g < total, owner, -1)


def _router(x, gate_w):
    T = x.shape[0]
    return pl.pallas_call(
        _router_body,
        out_shape=(
            jax.ShapeDtypeStruct((2, T), jnp.int32),
            jax.ShapeDtypeStruct((2, T), jnp.float32),
            jax.ShapeDtypeStruct((1, 32), jnp.int32),
        ),
    )(x, gate_w)


def _dispatch_body(x_hbm, pos_hbm, rw_hbm, xs_hbm, wgt_hbm,
                   pos_v, rw_v, gidx_v, wgt_v, rows_v, sem):
    T, D = x_hbm.shape
    nw = 32
    spt = NSLOT // nw
    ch = spt // 4
    wid = lax.axis_index("s") * 2 + lax.axis_index("c")

    pltpu.sync_copy(pos_hbm, pos_v)
    pltpu.sync_copy(rw_hbm, rw_v)

    def init(i, c):
        gidx_v[pl.ds(i * 16, 16)] = jnp.zeros((16,), jnp.int32)
        wgt_v[pl.ds(i * 16, 16)] = jnp.zeros((16,), jnp.float32)
        return c
    lax.fori_loop(0, NSLOT // 16, init, 0)

    def scat(j, c):
        tid = lax.iota(jnp.int32, 16) + j * 16
        for k in range(TOPK):
            p = pos_v[k, pl.ds(j * 16, 16)]
            plsc.store_scatter(gidx_v, [p], tid)
            plsc.store_scatter(wgt_v, [p], rw_v[k, pl.ds(j * 16, 16)])
        return c
    lax.fori_loop(0, T // 16, scat, 0)

    base = wid * spt
    pltpu.sync_copy(wgt_v.at[pl.ds(base, spt)], wgt_hbm.at[pl.ds(base, spt)])
    for c in range(4):
        off = base + c * ch
        pltpu.async_copy(x_hbm.at[gidx_v.at[pl.ds(off, ch)]], rows_v, sem).wait()
        pltpu.sync_copy(rows_v, xs_hbm.at[pl.ds(off, ch)])


def _dispatch(x, pos, rw):
    T, D = x.shape
    mesh = plsc.VectorSubcoreMesh(core_axis_name="c", subcore_axis_name="s")
    return pl.kernel(
        _dispatch_body,
        out_type=(
            jax.ShapeDtypeStruct((NSLOT, D), jnp.float32),
            jax.ShapeDtypeStruct((NSLOT,), jnp.float32),
        ),
        mesh=mesh,
        compiler_params=pltpu.CompilerParams(needs_layout_passes=False),
        scratch_types=[
            pltpu.VMEM((2, T), jnp.int32),
            pltpu.VMEM((2, T), jnp.float32),
            pltpu.VMEM((NSLOT,), jnp.int32),
            pltpu.VMEM((NSLOT,), jnp.float32),
            pltpu.VMEM((NSLOT // 32 // 4, D), jnp.float32),
            pltpu.SemaphoreType.DMA,
        ],
    )(x, pos, rw)


def _ffn_body(ex_ref, x_ref, w1_ref, w3_ref, w2_ref, wgt_ref, y_ref):
    @pl.when(ex_ref[pl.program_id(0)] >= 0)
    def _():
        x = x_ref[...]
        g = lax.dot_general(x, w1_ref[0], (((1,), (1,)), ((), ())),
                            preferred_element_type=jnp.float32)
        u = lax.dot_general(x, w3_ref[0], (((1,), (1,)), ((), ())),
                            preferred_element_type=jnp.float32)
        h = g * jax.nn.sigmoid(g) * u
        y = lax.dot_general(h, w2_ref[0], (((1,), (1,)), ((), ())),
                            preferred_element_type=jnp.float32)
        y_ref[...] = y * wgt_ref[0]


def _ffn(ex, xs, w1, w3, w2, wgt):
    E, F, D = w1.shape

    def wsel(g, ex_s):
        return (jnp.maximum(ex_s[g], 0), 0, 0)

    grid_spec = pltpu.PrefetchScalarGridSpec(
        num_scalar_prefetch=1,
        grid=(NUM_TILES,),
        in_specs=[
            pl.BlockSpec((TILE_M, D), lambda g, ex_s: (g, 0)),
            pl.BlockSpec((1, F, D), wsel),
            pl.BlockSpec((1, F, D), wsel),
            pl.BlockSpec((1, D, F), wsel),
            pl.BlockSpec((1, TILE_M, 1), lambda g, ex_s: (g, 0, 0)),
        ],
        out_specs=pl.BlockSpec((TILE_M, D), lambda g, ex_s: (g, 0)),
    )
    return pl.pallas_call(
        _ffn_body,
        grid_spec=grid_spec,
        out_shape=jax.ShapeDtypeStruct((NSLOT, D), jnp.float32),
    )(ex, xs, w1, w3, w2, wgt)


def _combine_body(y_hbm, pos_hbm, out_hbm, p0_v, p1_v, buf0, buf1, sem0, sem1):
    T = out_hbm.shape[0]
    D = out_hbm.shape[1]
    nw = 32
    tpt = T // nw
    ct = tpt // 2
    wid = lax.axis_index("s") * 2 + lax.axis_index("c")
    base = wid * tpt
    pltpu.sync_copy(pos_hbm.at[0, pl.ds(base, tpt)], p0_v)
    pltpu.sync_copy(pos_hbm.at[1, pl.ds(base, tpt)], p1_v)
    for c in range(2):
        pltpu.async_copy(y_hbm.at[p0_v.at[pl.ds(c * ct, ct)]], buf0, sem0).wait()
        pltpu.async_copy(y_hbm.at[p1_v.at[pl.ds(c * ct, ct)]], buf1, sem1).wait()

        def add(i, carry):
            r = lax.shift_right_logical(i, 6)
            col = lax.shift_left(jnp.bitwise_and(i, 63), 4)
            buf0[r, pl.ds(col, 16)] = buf0[r, pl.ds(col, 16)] + buf1[r, pl.ds(col, 16)]
            return carry
        lax.fori_loop(0, ct * (D // 16), add, 0, unroll=8)
        pltpu.sync_copy(buf0, out_hbm.at[pl.ds(base + c * ct, ct)])


def _combine(y, pos, T, D):
    mesh = plsc.VectorSubcoreMesh(core_axis_name="c", subcore_axis_name="s")
    return pl.kernel(
        _combine_body,
        out_type=jax.ShapeDtypeStruct((T, D), jnp.float32),
        mesh=mesh,
        compiler_params=pltpu.CompilerParams(needs_layout_passes=False),
        scratch_types=[
            pltpu.VMEM((T // 32,), jnp.int32),
            pltpu.VMEM((T // 32,), jnp.int32),
            pltpu.VMEM((T // 64, D), jnp.float32),
            pltpu.VMEM((T // 64, D), jnp.float32),
            pltpu.SemaphoreType.DMA,
            pltpu.SemaphoreType.DMA,
        ],
    )(y, pos)


def kernel(hidden_states, gate_w, w1, w3, w2):
    orig_shape = hidden_states.shape
    D = orig_shape[-1]
    x = hidden_states.reshape(-1, D)
    T = x.shape[0]
    pos, rw, ex = _router(x, gate_w)
    xs, wgt = _dispatch(x, pos, rw)
    y = _ffn(ex.reshape(32), xs, w1, w3, w2, wgt.reshape(NUM_TILES, TILE_M, 1))
    out = _combine(y, pos, T, D)
    return out.reshape(orig_shape)


# speedup vs baseline: 1.3408x; 1.0014x over previous
import functools

import jax
import jax.numpy as jnp
from jax import lax
from jax.experimental import pallas as pl
from jax.experimental.pallas import tpu as pltpu
from jax.experimental.pallas import tpu_sc as plsc

TOPK = 2
TILE_M = 256
TILE_SHIFT = 8
NUM_TILES = 24
NSLOT = NUM_TILES * TILE_M
SCAN_CHUNK = 512


def _router_body(x_ref, gw_ref, pos_ref, rw_ref, ex_ref):
    x = x_ref[...]
    gw = gw_ref[...]
    E = gw.shape[0]
    T = x.shape[0]
    logits = lax.dot_general(gw, x, (((1,), (1,)), ((), ())),
                             preferred_element_type=jnp.float32)
    row = lax.broadcasted_iota(jnp.int32, (E, T), 0)
    v0 = jnp.max(logits, axis=0, keepdims=True)
    a0 = jnp.min(jnp.where(logits == v0, row, E), axis=0, keepdims=True)
    masked = jnp.where(row == a0, -jnp.inf, logits)
    v1 = jnp.max(masked, axis=0, keepdims=True)
    a1 = jnp.min(jnp.where(masked == v1, row, E), axis=0, keepdims=True)
    d = jnp.exp(v1 - v0)
    w0 = 1.0 / (1.0 + d)
    w1 = d / (1.0 + d)

    oh0 = (row == a0).astype(jnp.float32)
    oh1 = (row == a1).astype(jnp.float32)

    C = SCAN_CHUNK
    ci = lax.broadcasted_iota(jnp.int32, (C, C), 0)
    cj = lax.broadcasted_iota(jnp.int32, (C, C), 1)
    upper_incl = (ci <= cj).astype(jnp.float32)
    carry = jnp.zeros((E, 1), jnp.float32)
    ranks = []
    for oh in (oh0, oh1):
        chunks = []
        for c in range(T // C):
            ohc = oh[:, c * C:(c + 1) * C]
            run = lax.dot_general(ohc, upper_incl, (((1,), (0,)), ((), ())),
                                  preferred_element_type=jnp.float32) + carry
            chunks.append(jnp.sum(run * ohc, axis=0, keepdims=True))
            carry = run[:, C - 1:C]
        ranks.append(jnp.concatenate(chunks, axis=1) - 1.0)
    counts = carry

    counts_i = counts.astype(jnp.int32)
    tiles = lax.shift_right_logical(counts_i + (TILE_M - 1), TILE_SHIFT)
    tiles_f = tiles.astype(jnp.float32)
    ei = lax.broadcasted_iota(jnp.int32, (E, E), 0)
    ej = lax.broadcasted_iota(jnp.int32, (E, E), 1)
    strict_lower = (ej < ei).astype(jnp.float32)
    tbase = lax.dot_general(strict_lower, tiles_f, (((1,), (0,)), ((), ())),
                            preferred_element_type=jnp.float32)
    pbase = tbase * float(TILE_M)

    pos0 = jnp.sum(oh0 * pbase, axis=0, keepdims=True) + ranks[0]
    pos1 = jnp.sum(oh1 * pbase, axis=0, keepdims=True) + ranks[1]
    pos_ref[0:1, :] = pos0.astype(jnp.int32)
    pos_ref[1:2, :] = pos1.astype(jnp.int32)
    rw_ref[0:1, :] = w0
    rw_ref[1:2, :] = w1

    g = lax.broadcasted_iota(jnp.int32, (1, 32), 1)
    owner = jnp.sum((tbase <= g.astype(jnp.float32)).astype(jnp.float32),
                    axis=0, keepdims=True).astype(jnp.int32) - 1
    total = jnp.sum(tiles_f).astype(jnp.int32)
    ex_ref[...] = jnp.where(g < total, owner, -1)


def _router(x, gate_w):
    T = x.shape[0]
    return pl.pallas_call(
        _router_body,
        out_shape=(
            jax.ShapeDtypeStruct((2, T), jnp.int32),
            jax.ShapeDtypeStruct((2, T), jnp.float32),
            jax.ShapeDtypeStruct((1, 32), jnp.int32),
        ),
    )(x, gate_w)


def _dispatch_body(x_hbm, pos_hbm, rw_hbm, xs_hbm, wgt_hbm,
                   pos_v, rw_v, gidx_v, wgt_v, rows0, rows1,
                   sg0, sg1, sw0, sw1):
    T, D = x_hbm.shape
    nw = 32
    spt = NSLOT // nw
    ch = spt // 4
    wid = lax.axis_index("s") * 2 + lax.axis_index("c")
    base = wid * spt

    pltpu.sync_copy(pos_hbm, pos_v)
    pltpu.sync_copy(rw_hbm, rw_v)

    def init(i, c):
        gidx_v[pl.ds(base + i * 16, 16)] = jnp.zeros((16,), jnp.int32)
        wgt_v[pl.ds(base + i * 16, 16)] = jnp.zeros((16,), jnp.float32)
        return c
    lax.fori_loop(0, spt // 16, init, 0)

    def scat(j, c):
        tid = lax.iota(jnp.int32, 16) + j * 16
        for k in range(TOPK):
            p = pos_v[k, pl.ds(j * 16, 16)]
            plsc.store_scatter(gidx_v, [p], tid)
            plsc.store_scatter(wgt_v, [p], rw_v[k, pl.ds(j * 16, 16)])
        return c
    lax.fori_loop(0, T // 16, scat, 0)

    pltpu.sync_copy(wgt_v.at[pl.ds(base, spt)], wgt_hbm.at[pl.ds(base, spt)])

    def gather(c, buf, sem):
        idx = gidx_v.at[pl.ds(base + c * ch, ch)]
        return pltpu.async_copy(x_hbm.at[idx], buf, sem)

    def put(c, buf, sem):
        return pltpu.async_copy(buf, xs_hbm.at[pl.ds(base + c * ch, ch)], sem)

    g0 = gather(0, rows0, sg0)
    g1 = gather(1, rows1, sg1)
    g0.wait()
    w0 = put(0, rows0, sw0)
    g1.wait()
    w1 = put(1, rows1, sw1)
    w0.wait()
    g2 = gather(2, rows0, sg0)
    w1.wait()
    g3 = gather(3, rows1, sg1)
    g2.wait()
    w2 = put(2, rows0, sw0)
    g3.wait()
    w3 = put(3, rows1, sw1)
    w2.wait()
    w3.wait()


def _dispatch(x, pos, rw):
    T, D = x.shape
    mesh = plsc.VectorSubcoreMesh(core_axis_name="c", subcore_axis_name="s")
    ch = NSLOT // 32 // 4
    return pl.kernel(
        _dispatch_body,
        out_type=(
            jax.ShapeDtypeStruct((NSLOT, D), jnp.float32),
            jax.ShapeDtypeStruct((NSLOT,), jnp.float32),
        ),
        mesh=mesh,
        compiler_params=pltpu.CompilerParams(needs_layout_passes=False),
        scratch_types=[
            pltpu.VMEM((2, T), jnp.int32),
            pltpu.VMEM((2, T), jnp.float32),
            pltpu.VMEM((NSLOT,), jnp.int32),
            pltpu.VMEM((NSLOT,), jnp.float32),
            pltpu.VMEM((ch, D), jnp.float32),
            pltpu.VMEM((ch, D), jnp.float32),
            pltpu.SemaphoreType.DMA,
            pltpu.SemaphoreType.DMA,
            pltpu.SemaphoreType.DMA,
            pltpu.SemaphoreType.DMA,
        ],
    )(x, pos, rw)


def _ffn_body(ex_ref, x_ref, w1_ref, w3_ref, w2_ref, wgt_ref, y_ref):
    @pl.when(ex_ref[pl.program_id(0)] >= 0)
    def _():
        x = x_ref[...]
        g = lax.dot_general(x, w1_ref[0], (((1,), (1,)), ((), ())),
                            preferred_element_type=jnp.float32)
        u = lax.dot_general(x, w3_ref[0], (((1,), (1,)), ((), ())),
                            preferred_element_type=jnp.float32)
        h = g * jax.nn.sigmoid(g) * u
        y = lax.dot_general(h, w2_ref[0], (((1,), (1,)), ((), ())),
                            preferred_element_type=jnp.float32)
        y_ref[...] = y * wgt_ref[0]


def _ffn(ex, xs, w1, w3, w2, wgt):
    E, F, D = w1.shape

    def wsel(g, ex_s):
        return (jnp.maximum(ex_s[g], 0), 0, 0)

    grid_spec = pltpu.PrefetchScalarGridSpec(
        num_scalar_prefetch=1,
        grid=(NUM_TILES,),
        in_specs=[
            pl.BlockSpec((TILE_M, D), lambda g, ex_s: (g, 0)),
            pl.BlockSpec((1, F, D), wsel),
            pl.BlockSpec((1, F, D), wsel),
            pl.BlockSpec((1, D, F), wsel),
            pl.BlockSpec((1, TILE_M, 1), lambda g, ex_s: (g, 0, 0)),
        ],
        out_specs=pl.BlockSpec((TILE_M, D), lambda g, ex_s: (g, 0)),
    )
    return pl.pallas_call(
        _ffn_body,
        grid_spec=grid_spec,
        out_shape=jax.ShapeDtypeStruct((NSLOT, D), jnp.float32),
    )(ex, xs, w1, w3, w2, wgt)


def _combine_body(y_hbm, pos_hbm, out_hbm, p0_v, p1_v, buf0, buf1, sem0, sem1):
    T = out_hbm.shape[0]
    D = out_hbm.shape[1]
    nw = 32
    tpt = T // nw
    ct = tpt // 2
    wid = lax.axis_index("s") * 2 + lax.axis_index("c")
    base = wid * tpt
    pltpu.sync_copy(pos_hbm.at[0, pl.ds(base, tpt)], p0_v)
    pltpu.sync_copy(pos_hbm.at[1, pl.ds(base, tpt)], p1_v)
    for c in range(2):
        pltpu.async_copy(y_hbm.at[p0_v.at[pl.ds(c * ct, ct)]], buf0, sem0).wait()
        pltpu.async_copy(y_hbm.at[p1_v.at[pl.ds(c * ct, ct)]], buf1, sem1).wait()

        def add(i, carry):
            r = lax.shift_right_logical(i, 6)
            col = lax.shift_left(jnp.bitwise_and(i, 63), 4)
            buf0[r, pl.ds(col, 16)] = buf0[r, pl.ds(col, 16)] + buf1[r, pl.ds(col, 16)]
            return carry
        lax.fori_loop(0, ct * (D // 16), add, 0, unroll=8)
        pltpu.sync_copy(buf0, out_hbm.at[pl.ds(base + c * ct, ct)])


def _combine(y, pos, T, D):
    mesh = plsc.VectorSubcoreMesh(core_axis_name="c", subcore_axis_name="s")
    return pl.kernel(
        _combine_body,
        out_type=jax.ShapeDtypeStruct((T, D), jnp.float32),
        mesh=mesh,
        compiler_params=pltpu.CompilerParams(needs_layout_passes=False),
        scratch_types=[
            pltpu.VMEM((T // 32,), jnp.int32),
            pltpu.VMEM((T // 32,), jnp.int32),
            pltpu.VMEM((T // 64, D), jnp.float32),
            pltpu.VMEM((T // 64, D), jnp.float32),
            pltpu.SemaphoreType.DMA,
            pltpu.SemaphoreType.DMA,
        ],
    )(y, pos)


def kernel(hidden_states, gate_w, w1, w3, w2):
    orig_shape = hidden_states.shape
    D = orig_shape[-1]
    x = hidden_states.reshape(-1, D)
    T = x.shape[0]
    pos, rw, ex = _router(x, gate_w)
    xs, wgt = _dispatch(x, pos, rw)
    y = _ffn(ex.reshape(32), xs, w1, w3, w2, wgt.reshape(NUM_TILES, TILE_M, 1))
    out = _combine(y, pos, T, D)
    return out.reshape(orig_shape)


# speedup vs baseline: 1.3429x; 1.0015x over previous
import functools

import jax
import jax.numpy as jnp
from jax import lax
from jax.experimental import pallas as pl
from jax.experimental.pallas import tpu as pltpu
from jax.experimental.pallas import tpu_sc as plsc

TOPK = 2
TILE_M = 256
TILE_SHIFT = 8
NUM_TILES = 24
NSLOT = NUM_TILES * TILE_M
SCAN_CHUNK = 512


def _router_body(x_ref, gw_ref, pos_ref, rw_ref, ex_ref):
    x = x_ref[...]
    gw = gw_ref[...]
    E = gw.shape[0]
    T = x.shape[0]
    logits = lax.dot_general(gw, x, (((1,), (1,)), ((), ())),
                             preferred_element_type=jnp.float32)
    row = lax.broadcasted_iota(jnp.int32, (E, T), 0)
    v0 = jnp.max(logits, axis=0, keepdims=True)
    a0 = jnp.min(jnp.where(logits == v0, row, E), axis=0, keepdims=True)
    masked = jnp.where(row == a0, -jnp.inf, logits)
    v1 = jnp.max(masked, axis=0, keepdims=True)
    a1 = jnp.min(jnp.where(masked == v1, row, E), axis=0, keepdims=True)
    d = jnp.exp(v1 - v0)
    w0 = 1.0 / (1.0 + d)
    w1 = d / (1.0 + d)

    oh0 = (row == a0).astype(jnp.float32)
    oh1 = (row == a1).astype(jnp.float32)

    C = SCAN_CHUNK
    ci = lax.broadcasted_iota(jnp.int32, (C, C), 0)
    cj = lax.broadcasted_iota(jnp.int32, (C, C), 1)
    upper_incl = (ci <= cj).astype(jnp.float32)
    carry = jnp.zeros((E, 1), jnp.float32)
    ranks = []
    for oh in (oh0, oh1):
        chunks = []
        for c in range(T // C):
            ohc = oh[:, c * C:(c + 1) * C]
            run = lax.dot_general(ohc, upper_incl, (((1,), (0,)), ((), ())),
                                  preferred_element_type=jnp.float32) + carry
            chunks.append(jnp.sum(run * ohc, axis=0, keepdims=True))
            carry = run[:, C - 1:C]
        ranks.append(jnp.concatenate(chunks, axis=1) - 1.0)
    counts = carry

    counts_i = counts.astype(jnp.int32)
    tiles = lax.shift_right_logical(counts_i + (TILE_M - 1), TILE_SHIFT)
    tiles_f = tiles.astype(jnp.float32)
    ei = lax.broadcasted_iota(jnp.int32, (E, E), 0)
    ej = lax.broadcasted_iota(jnp.int32, (E, E), 1)
    strict_lower = (ej < ei).astype(jnp.float32)
    tbase = lax.dot_general(strict_lower, tiles_f, (((1,), (0,)), ((), ())),
                            preferred_element_type=jnp.float32)
    pbase = tbase * float(TILE_M)

    pos0 = jnp.sum(oh0 * pbase, axis=0, keepdims=True) + ranks[0]
    pos1 = jnp.sum(oh1 * pbase, axis=0, keepdims=True) + ranks[1]
    pos_ref[0:1, :] = pos0.astype(jnp.int32)
    pos_ref[1:2, :] = pos1.astype(jnp.int32)
    rw_ref[0:1, :] = w0
    rw_ref[1:2, :] = w1

    g = lax.broadcasted_iota(jnp.int32, (1, 32), 1)
    owner = jnp.sum((tbase <= g.astype(jnp.float32)).astype(jnp.float32),
                    axis=0, keepdims=True).astype(jnp.int32) - 1
    total = jnp.sum(tiles_f).astype(jnp.int32)
    ex_ref[...] = jnp.where(g < total, owner, -1)


def _router(x, gate_w):
    T = x.shape[0]
    return pl.pallas_call(
        _router_body,
        out_shape=(
            jax.ShapeDtypeStruct((2, T), jnp.int32),
            jax.ShapeDtypeStruct((2, T), jnp.float32),
            jax.ShapeDtypeStruct((1, 32), jnp.int32),
        ),
    )(x, gate_w)


def _dispatch_body(x_hbm, pos_hbm, rw_hbm, xs_hbm, wgt_hbm,
                   pos_v, rw_v, gidx_v, wgt_v, rows0, rows1,
                   sg0, sg1, sw0, sw1):
    T, D = x_hbm.shape
    nw = 32
    spt = NSLOT // nw
    ch = spt // 4
    wid = lax.axis_index("s") * 2 + lax.axis_index("c")
    base = wid * spt

    with jax.named_scope("disp_meta"):
        pltpu.sync_copy(pos_hbm, pos_v)
        pltpu.sync_copy(rw_hbm, rw_v)

        def init(i, c):
            gidx_v[pl.ds(base + i * 16, 16)] = jnp.zeros((16,), jnp.int32)
            wgt_v[pl.ds(base + i * 16, 16)] = jnp.zeros((16,), jnp.float32)
            return c
        lax.fori_loop(0, spt // 16, init, 0)

        def scat(j, c):
            tid = lax.iota(jnp.int32, 16) + j * 16
            for k in range(TOPK):
                p = pos_v[k, pl.ds(j * 16, 16)]
                plsc.store_scatter(gidx_v, [p], tid)
                plsc.store_scatter(wgt_v, [p], rw_v[k, pl.ds(j * 16, 16)])
            return c
        lax.fori_loop(0, T // 16, scat, 0)

        pltpu.sync_copy(wgt_v.at[pl.ds(base, spt)], wgt_hbm.at[pl.ds(base, spt)])

    def gather(c, buf, sem):
        idx = gidx_v.at[pl.ds(base + c * ch, ch)]
        return pltpu.async_copy(x_hbm.at[idx], buf, sem)

    def put(c, buf, sem):
        return pltpu.async_copy(buf, xs_hbm.at[pl.ds(base + c * ch, ch)], sem)

    with jax.named_scope("disp_gather"):
        g0 = gather(0, rows0, sg0)
        g1 = gather(1, rows1, sg1)
        g0.wait()
        w0 = put(0, rows0, sw0)
        g1.wait()
        w1 = put(1, rows1, sw1)
        w0.wait()
        g2 = gather(2, rows0, sg0)
        w1.wait()
        g3 = gather(3, rows1, sg1)
        g2.wait()
        w2 = put(2, rows0, sw0)
        g3.wait()
        w3 = put(3, rows1, sw1)
        w2.wait()
        w3.wait()


def _dispatch(x, pos, rw):
    T, D = x.shape
    mesh = plsc.VectorSubcoreMesh(core_axis_name="c", subcore_axis_name="s")
    ch = NSLOT // 32 // 4
    return pl.kernel(
        _dispatch_body,
        out_type=(
            jax.ShapeDtypeStruct((NSLOT, D), jnp.float32),
            jax.ShapeDtypeStruct((NSLOT,), jnp.float32),
        ),
        mesh=mesh,
        compiler_params=pltpu.CompilerParams(needs_layout_passes=False),
        scratch_types=[
            pltpu.VMEM((2, T), jnp.int32),
            pltpu.VMEM((2, T), jnp.float32),
            pltpu.VMEM((NSLOT,), jnp.int32),
            pltpu.VMEM((NSLOT,), jnp.float32),
            pltpu.VMEM((ch, D), jnp.float32),
            pltpu.VMEM((ch, D), jnp.float32),
            pltpu.SemaphoreType.DMA,
            pltpu.SemaphoreType.DMA,
            pltpu.SemaphoreType.DMA,
            pltpu.SemaphoreType.DMA,
        ],
    )(x, pos, rw)


def _ffn_body(ex_ref, x_ref, w1_ref, w3_ref, w2_ref, wgt_ref, y_ref):
    @pl.when(ex_ref[pl.program_id(0)] >= 0)
    def _():
        x = x_ref[...]
        g = lax.dot_general(x, w1_ref[0], (((1,), (1,)), ((), ())),
                            preferred_element_type=jnp.float32)
        u = lax.dot_general(x, w3_ref[0], (((1,), (1,)), ((), ())),
                            preferred_element_type=jnp.float32)
        h = g * jax.nn.sigmoid(g) * u
        y = lax.dot_general(h, w2_ref[0], (((1,), (1,)), ((), ())),
                            preferred_element_type=jnp.float32)
        y_ref[...] = y * wgt_ref[0]


def _ffn(ex, xs, w1, w3, w2, wgt):
    E, F, D = w1.shape

    def wsel(g, ex_s):
        return (jnp.maximum(ex_s[g], 0), 0, 0)

    grid_spec = pltpu.PrefetchScalarGridSpec(
        num_scalar_prefetch=1,
        grid=(NUM_TILES,),
        in_specs=[
            pl.BlockSpec((TILE_M, D), lambda g, ex_s: (g, 0)),
            pl.BlockSpec((1, F, D), wsel),
            pl.BlockSpec((1, F, D), wsel),
            pl.BlockSpec((1, D, F), wsel),
            pl.BlockSpec((1, TILE_M, 1), lambda g, ex_s: (g, 0, 0)),
        ],
        out_specs=pl.BlockSpec((TILE_M, D), lambda g, ex_s: (g, 0)),
    )
    return pl.pallas_call(
        _ffn_body,
        grid_spec=grid_spec,
        out_shape=jax.ShapeDtypeStruct((NSLOT, D), jnp.float32),
    )(ex, xs, w1, w3, w2, wgt)


def _combine_body(y_hbm, pos_hbm, out_hbm, p0_v, p1_v, buf0, buf1, sem0, sem1):
    T = out_hbm.shape[0]
    D = out_hbm.shape[1]
    nw = 32
    tpt = T // nw
    ct = tpt // 2
    wid = lax.axis_index("s") * 2 + lax.axis_index("c")
    base = wid * tpt
    pltpu.sync_copy(pos_hbm.at[0, pl.ds(base, tpt)], p0_v)
    pltpu.sync_copy(pos_hbm.at[1, pl.ds(base, tpt)], p1_v)
    for c in range(2):
        pltpu.async_copy(y_hbm.at[p0_v.at[pl.ds(c * ct, ct)]], buf0, sem0).wait()
        pltpu.async_copy(y_hbm.at[p1_v.at[pl.ds(c * ct, ct)]], buf1, sem1).wait()

        def add(i, carry):
            r = lax.shift_right_logical(i, 6)
            col = lax.shift_left(jnp.bitwise_and(i, 63), 4)
            buf0[r, pl.ds(col, 16)] = buf0[r, pl.ds(col, 16)] + buf1[r, pl.ds(col, 16)]
            return carry
        lax.fori_loop(0, ct * (D // 16), add, 0, unroll=8)
        pltpu.sync_copy(buf0, out_hbm.at[pl.ds(base + c * ct, ct)])


def _combine(y, pos, T, D):
    mesh = plsc.VectorSubcoreMesh(core_axis_name="c", subcore_axis_name="s")
    return pl.kernel(
        _combine_body,
        out_type=jax.ShapeDtypeStruct((T, D), jnp.float32),
        mesh=mesh,
        compiler_params=pltpu.CompilerParams(needs_layout_passes=False),
        scratch_types=[
            pltpu.VMEM((T // 32,), jnp.int32),
            pltpu.VMEM((T // 32,), jnp.int32),
            pltpu.VMEM((T // 64, D), jnp.float32),
            pltpu.VMEM((T // 64, D), jnp.float32),
            pltpu.SemaphoreType.DMA,
            pltpu.SemaphoreType.DMA,
        ],
    )(y, pos)


def kernel(hidden_states, gate_w, w1, w3, w2):
    orig_shape = hidden_states.shape
    D = orig_shape[-1]
    x = hidden_states.reshape(-1, D)
    T = x.shape[0]
    pos, rw, ex = _router(x, gate_w)
    xs, wgt = _dispatch(x, pos, rw)
    y = _ffn(ex.reshape(32), xs, w1, w3, w2, wgt.reshape(NUM_TILES, TILE_M, 1))
    out = _combine(y, pos, T, D)
    return out.reshape(orig_shape)


# speedup vs baseline: 2.3605x; 1.7578x over previous
import functools

import jax
import jax.numpy as jnp
from jax import lax
from jax.experimental import pallas as pl
from jax.experimental.pallas import tpu as pltpu
from jax.experimental.pallas import tpu_sc as plsc

TOPK = 2
TILE_M = 256
TILE_SHIFT = 8
NUM_TILES = 24
NSLOT = NUM_TILES * TILE_M
SCAN_CHUNK = 512


def _router_body(x_ref, gw_ref, pos_ref, rw_ref, ex_ref, used_ref):
    x = x_ref[...]
    gw = gw_ref[...]
    E = gw.shape[0]
    T = x.shape[0]
    logits = lax.dot_general(gw, x, (((1,), (1,)), ((), ())),
                             preferred_element_type=jnp.float32)
    row = lax.broadcasted_iota(jnp.int32, (E, T), 0)
    v0 = jnp.max(logits, axis=0, keepdims=True)
    a0 = jnp.min(jnp.where(logits == v0, row, E), axis=0, keepdims=True)
    masked = jnp.where(row == a0, -jnp.inf, logits)
    v1 = jnp.max(masked, axis=0, keepdims=True)
    a1 = jnp.min(jnp.where(masked == v1, row, E), axis=0, keepdims=True)
    d = jnp.exp(v1 - v0)
    w0 = 1.0 / (1.0 + d)
    w1 = d / (1.0 + d)

    oh0 = (row == a0).astype(jnp.float32)
    oh1 = (row == a1).astype(jnp.float32)

    C = SCAN_CHUNK
    ci = lax.broadcasted_iota(jnp.int32, (C, C), 0)
    cj = lax.broadcasted_iota(jnp.int32, (C, C), 1)
    upper_incl = (ci <= cj).astype(jnp.float32)
    carry = jnp.zeros((E, 1), jnp.float32)
    ranks = []
    for oh in (oh0, oh1):
        chunks = []
        for c in range(T // C):
            ohc = oh[:, c * C:(c + 1) * C]
            run = lax.dot_general(ohc, upper_incl, (((1,), (0,)), ((), ())),
                                  preferred_element_type=jnp.float32) + carry
            chunks.append(jnp.sum(run * ohc, axis=0, keepdims=True))
            carry = run[:, C - 1:C]
        ranks.append(jnp.concatenate(chunks, axis=1) - 1.0)
    counts = carry

    counts_i = counts.astype(jnp.int32)
    tiles = lax.shift_right_logical(counts_i + (TILE_M - 1), TILE_SHIFT)
    tiles_f = tiles.astype(jnp.float32)
    ei = lax.broadcasted_iota(jnp.int32, (E, E), 0)
    ej = lax.broadcasted_iota(jnp.int32, (E, E), 1)
    strict_lower = (ej < ei).astype(jnp.float32)
    tbase = lax.dot_general(strict_lower, tiles_f, (((1,), (0,)), ((), ())),
                            preferred_element_type=jnp.float32)
    pbase = tbase * float(TILE_M)

    pos0 = jnp.sum(oh0 * pbase, axis=0, keepdims=True) + ranks[0]
    pos1 = jnp.sum(oh1 * pbase, axis=0, keepdims=True) + ranks[1]
    pos_ref[0:1, :] = pos0.astype(jnp.int32)
    pos_ref[1:2, :] = pos1.astype(jnp.int32)
    rw_ref[0:1, :] = w0
    rw_ref[1:2, :] = w1

    g = lax.broadcasted_iota(jnp.int32, (1, 32), 1)
    owner = jnp.sum((tbase <= g.astype(jnp.float32)).astype(jnp.float32),
                    axis=0, keepdims=True).astype(jnp.int32) - 1
    total = jnp.sum(tiles_f).astype(jnp.int32)
    ex_ref[...] = jnp.where(g < total, owner, -1)
    used_ref[...] = jnp.zeros((1, 16), jnp.int32) + total * TILE_M


def _router(x, gate_w):
    T = x.shape[0]
    return pl.pallas_call(
        _router_body,
        out_shape=(
            jax.ShapeDtypeStruct((2, T), jnp.int32),
            jax.ShapeDtypeStruct((2, T), jnp.float32),
            jax.ShapeDtypeStruct((1, 32), jnp.int32),
            jax.ShapeDtypeStruct((1, 16), jnp.int32),
        ),
    )(x, gate_w)


def _dispatch_body(x_hbm, pos_hbm, rw_hbm, used_hbm, xs_hbm, wgt_hbm,
                   pos_v, rw_v, used_v, gidx_v, wgt_v, rows0, rows1,
                   sg0, sg1, sw0, sw1):
    T, D = x_hbm.shape
    nw = 32
    spt = NSLOT // nw
    ch = spt // 4
    wid = lax.axis_index("s") * 2 + lax.axis_index("c")
    base = wid * spt

    with jax.named_scope("disp_meta"):
        pltpu.sync_copy(pos_hbm, pos_v)
        pltpu.sync_copy(rw_hbm, rw_v)
        pltpu.sync_copy(used_hbm, used_v)

        def init(i, c):
            off = base + i * 16
            gidx_v[pl.ds(off, 16)] = jnp.bitwise_and(
                lax.iota(jnp.int32, 16) + off, T - 1)
            wgt_v[pl.ds(off, 16)] = jnp.zeros((16,), jnp.float32)
            return c
        lax.fori_loop(0, spt // 16, init, 0)

        def scat(j, c):
            tid = lax.iota(jnp.int32, 16) + j * 16
            for k in range(TOPK):
                p = pos_v[k, pl.ds(j * 16, 16)]
                plsc.store_scatter(gidx_v, [p], tid)
                plsc.store_scatter(wgt_v, [p], rw_v[k, pl.ds(j * 16, 16)])
            return c
        lax.fori_loop(0, T // 16, scat, 0)

        pltpu.sync_copy(wgt_v.at[pl.ds(base, spt)], wgt_hbm.at[pl.ds(base, spt)])

    used = used_v[0, pl.ds(0, 16)][0]

    def copy_in(c, buf, sem):
        idx = gidx_v.at[pl.ds(base + c * ch, ch)]
        return pltpu.make_async_copy(x_hbm.at[idx], buf, sem)

    def copy_out(c, buf, sem):
        return pltpu.make_async_copy(buf, xs_hbm.at[pl.ds(base + c * ch, ch)], sem)

    with jax.named_scope("disp_gather"):
        live = [base + c * ch < used for c in range(4)]
        pl.when(live[0])(lambda: copy_in(0, rows0, sg0).start())
        pl.when(live[1])(lambda: copy_in(1, rows1, sg1).start())

        def drain0():
            copy_in(0, rows0, sg0).wait()
            copy_out(0, rows0, sw0).start()
        pl.when(live[0])(drain0)

        def drain1():
            copy_in(1, rows1, sg1).wait()
            copy_out(1, rows1, sw1).start()
        pl.when(live[1])(drain1)

        def next2():
            copy_out(0, rows0, sw0).wait()
        pl.when(live[0])(next2)
        pl.when(live[2])(lambda: copy_in(2, rows0, sg0).start())
        pl.when(live[1])(lambda: copy_out(1, rows1, sw1).wait())
        pl.when(live[3])(lambda: copy_in(3, rows1, sg1).start())

        def drain2():
            copy_in(2, rows0, sg0).wait()
            copy_out(2, rows0, sw0).start()
        pl.when(live[2])(drain2)

        def drain3():
            copy_in(3, rows1, sg1).wait()
            copy_out(3, rows1, sw1).start()
        pl.when(live[3])(drain3)
        pl.when(live[2])(lambda: copy_out(2, rows0, sw0).wait())
        pl.when(live[3])(lambda: copy_out(3, rows1, sw1).wait())


def _dispatch(x, pos, rw, used):
    T, D = x.shape
    mesh = plsc.VectorSubcoreMesh(core_axis_name="c", subcore_axis_name="s")
    ch = NSLOT // 32 // 4
    return pl.kernel(
        _dispatch_body,
        out_type=(
            jax.ShapeDtypeStruct((NSLOT, D), jnp.float32),
            jax.ShapeDtypeStruct((NSLOT,), jnp.float32),
        ),
        mesh=mesh,
        compiler_params=pltpu.CompilerParams(needs_layout_passes=False),
        scratch_types=[
            pltpu.VMEM((2, T), jnp.int32),
            pltpu.VMEM((2, T), jnp.float32),
            pltpu.VMEM((1, 16), jnp.int32),
            pltpu.VMEM((NSLOT,), jnp.int32),
            pltpu.VMEM((NSLOT,), jnp.float32),
            pltpu.VMEM((ch, D), jnp.float32),
            pltpu.VMEM((ch, D), jnp.float32),
            pltpu.SemaphoreType.DMA,
            pltpu.SemaphoreType.DMA,
            pltpu.SemaphoreType.DMA,
            pltpu.SemaphoreType.DMA,
        ],
    )(x, pos, rw, used)


def _ffn_body(ex_ref, x_ref, w1_ref, w3_ref, w2_ref, wgt_ref, y_ref):
    @pl.when(ex_ref[pl.program_id(0)] >= 0)
    def _():
        x = x_ref[...]
        g = lax.dot_general(x, w1_ref[0], (((1,), (1,)), ((), ())),
                            preferred_element_type=jnp.float32)
        u = lax.dot_general(x, w3_ref[0], (((1,), (1,)), ((), ())),
                            preferred_element_type=jnp.float32)
        h = g * jax.nn.sigmoid(g) * u
        y = lax.dot_general(h, w2_ref[0], (((1,), (1,)), ((), ())),
                            preferred_element_type=jnp.float32)
        y_ref[...] = y * wgt_ref[0]


def _ffn(ex, xs, w1, w3, w2, wgt):
    E, F, D = w1.shape

    def wsel(g, ex_s):
        return (jnp.maximum(ex_s[g], 0), 0, 0)

    grid_spec = pltpu.PrefetchScalarGridSpec(
        num_scalar_prefetch=1,
        grid=(NUM_TILES,),
        in_specs=[
            pl.BlockSpec((TILE_M, D), lambda g, ex_s: (g, 0)),
            pl.BlockSpec((1, F, D), wsel),
            pl.BlockSpec((1, F, D), wsel),
            pl.BlockSpec((1, D, F), wsel),
            pl.BlockSpec((1, TILE_M, 1), lambda g, ex_s: (g, 0, 0)),
        ],
        out_specs=pl.BlockSpec((TILE_M, D), lambda g, ex_s: (g, 0)),
    )
    return pl.pallas_call(
        _ffn_body,
        grid_spec=grid_spec,
        out_shape=jax.ShapeDtypeStruct((NSLOT, D), jnp.float32),
    )(ex, xs, w1, w3, w2, wgt)


def _combine_body(y_hbm, pos_hbm, out_hbm, p0_v, p1_v, buf0, buf1, sem0, sem1):
    T = out_hbm.shape[0]
    D = out_hbm.shape[1]
    nw = 32
    tpt = T // nw
    ct = tpt // 2
    wid = lax.axis_index("s") * 2 + lax.axis_index("c")
    base = wid * tpt
    pltpu.sync_copy(pos_hbm.at[0, pl.ds(base, tpt)], p0_v)
    pltpu.sync_copy(pos_hbm.at[1, pl.ds(base, tpt)], p1_v)
    for c in range(2):
        pltpu.async_copy(y_hbm.at[p0_v.at[pl.ds(c * ct, ct)]], buf0, sem0).wait()
        pltpu.async_copy(y_hbm.at[p1_v.at[pl.ds(c * ct, ct)]], buf1, sem1).wait()

        def add(i, carry):
            r = lax.shift_right_logical(i, 6)
            col = lax.shift_left(jnp.bitwise_and(i, 63), 4)
            buf0[r, pl.ds(col, 16)] = buf0[r, pl.ds(col, 16)] + buf1[r, pl.ds(col, 16)]
            return carry
        lax.fori_loop(0, ct * (D // 16), add, 0, unroll=8)
        pltpu.sync_copy(buf0, out_hbm.at[pl.ds(base + c * ct, ct)])


def _combine(y, pos, T, D):
    mesh = plsc.VectorSubcoreMesh(core_axis_name="c", subcore_axis_name="s")
    return pl.kernel(
        _combine_body,
        out_type=jax.ShapeDtypeStruct((T, D), jnp.float32),
        mesh=mesh,
        compiler_params=pltpu.CompilerParams(needs_layout_passes=False),
        scratch_types=[
            pltpu.VMEM((T // 32,), jnp.int32),
            pltpu.VMEM((T // 32,), jnp.int32),
            pltpu.VMEM((T // 64, D), jnp.float32),
            pltpu.VMEM((T // 64, D), jnp.float32),
            pltpu.SemaphoreType.DMA,
            pltpu.SemaphoreType.DMA,
        ],
    )(y, pos)


def kernel(hidden_states, gate_w, w1, w3, w2):
    orig_shape = hidden_states.shape
    D = orig_shape[-1]
    x = hidden_states.reshape(-1, D)
    T = x.shape[0]
    pos, rw, ex, used = _router(x, gate_w)
    xs, wgt = _dispatch(x, pos, rw, used)
    y = _ffn(ex.reshape(32), xs, w1, w3, w2, wgt.reshape(NUM_TILES, TILE_M, 1))
    out = _combine(y, pos, T, D)
    return out.reshape(orig_shape)
